# baseline (speedup 1.0000x reference)
import functools

import jax
import jax.numpy as jnp
from jax import lax
from jax.experimental import pallas as pl
from jax.experimental.pallas import tpu as pltpu
from jax.experimental.pallas import tpu_sc as plsc

N = 10000
E = 320000
D = 128
N2 = 10240
NW = 32
CHUNK = 128
NCH = 80
EP = NW * CHUNK * NCH
TOTC = EP // CHUNK
K0 = 80
K1 = 80
KPAIR = K0 + K1
KMAX = max(K0, K1)
SLAB = N2 // 16
BLK = 1024
GRID = N2 // BLK

_sc_mesh = plsc.VectorSubcoreMesh(core_axis_name="c", subcore_axis_name="s")


@functools.partial(
    pl.kernel, mesh=_sc_mesh,
    out_type=jax.ShapeDtypeStruct((2, N2), jnp.float32),
    scratch_types=[
        pltpu.VMEM((NCH, CHUNK), jnp.int32),
        pltpu.VMEM((CHUNK,), jnp.float32),
        pltpu.VMEM_SHARED((N2,), jnp.float32),
    ],
)
def _sc_deg(dst_hbm, zeros1_hbm, out_hbm, dst_v, ones_v, acc_sp):
    c = lax.axis_index("c")
    s = lax.axis_index("s")
    base = (s * 2 + c) * NCH
    pltpu.sync_copy(zeros1_hbm.at[pl.ds(s * SLAB, SLAB)],
                    acc_sp.at[pl.ds(s * SLAB, SLAB)])
    for l in range(CHUNK // 16):
        ones_v[pl.ds(l * 16, 16)] = jnp.ones((16,), jnp.float32)
    pltpu.sync_copy(dst_hbm.at[pl.ds(base, NCH)], dst_v)
    plsc.subcore_barrier()

    def body(j, carry):
        pltpu.sync_copy(ones_v, acc_sp.at[dst_v.at[j]], add=True)
        return carry

    lax.fori_loop(0, NCH, body, 0)
    plsc.subcore_barrier()
    pltpu.sync_copy(acc_sp.at[pl.ds(s * SLAB, SLAB)],
                    out_hbm.at[c, pl.ds(s * SLAB, SLAB)])


@functools.partial(
    pl.kernel, mesh=_sc_mesh,
    out_type=jax.ShapeDtypeStruct((2, N2, D), jnp.float32),
    scratch_types=[
        pltpu.VMEM((KMAX // 2 + 1, CHUNK), jnp.int32),
        pltpu.VMEM((KMAX // 2, CHUNK), jnp.int32),
        pltpu.VMEM((CHUNK, D), jnp.float32),
        pltpu.VMEM((CHUNK, D), jnp.float32),
        pltpu.VMEM_SHARED((N2, D), jnp.float32),
        pltpu.SemaphoreType.DMA,
        pltpu.SemaphoreType.DMA,
    ],
)
def _sc_scatter(q_hbm, src_hbm, dst_hbm, zeros2_hbm, out_hbm,
                src_v, dst_v, rows0, rows1, acc_sp, semg, sems):
    c = lax.axis_index("c")
    s = lax.axis_index("s")
    base = s * KPAIR + c * K0
    pltpu.sync_copy(zeros2_hbm.at[pl.ds(s * SLAB, SLAB)],
                    acc_sp.at[pl.ds(s * SLAB, SLAB)])
    plsc.subcore_barrier()

    def run(K):
        H = K // 2
        for l in range(CHUNK // 16):
            src_v[H, pl.ds(l * 16, 16)] = jnp.zeros((16,), jnp.int32)
        for p in range(2):
            pltpu.sync_copy(src_hbm.at[pl.ds(base + p * H, H)],
                            src_v.at[pl.ds(0, H)])
            pltpu.sync_copy(dst_hbm.at[pl.ds(base + p * H, H)], dst_v)
            pltpu.sync_copy(q_hbm.at[src_v.at[0]], rows0)

            def body(i, carry):
                j = i * 2
                g1 = pltpu.async_copy(q_hbm.at[src_v.at[j + 1]], rows1, semg)
                s0 = pltpu.async_copy(rows0, acc_sp.at[dst_v.at[j]], sems,
                                      add=True)
                g1.wait()
                s0.wait()
                g2 = pltpu.async_copy(q_hbm.at[src_v.at[j + 2]], rows0, semg)
                s1 = pltpu.async_copy(rows1, acc_sp.at[dst_v.at[j + 1]], sems,
                                      add=True)
                g2.wait()
                s1.wait()
                return carry

            lax.fori_loop(0, H // 2, body, 0)

    pl.when(c == 0)(lambda: run(K0))
    pl.when(c == 1)(lambda: run(K1))
    plsc.subcore_barrier()
    pltpu.sync_copy(acc_sp.at[pl.ds(s * SLAB, SLAB)],
                    out_hbm.at[c, pl.ds(s * SLAB, SLAB)])


def _dinv_body(deg_ref, out_ref):
    d = deg_ref[0] + deg_ref[1] + 1.0
    out_ref[...] = lax.rsqrt(d)


_tc_dinv = pl.pallas_call(
    _dinv_body,
    out_shape=jax.ShapeDtypeStruct((N2 // D, D), jnp.float32),
)


def _first_body(x_ref, w_ref, dinv_ref, out_ref):
    hw = jnp.dot(x_ref[...], w_ref[...], preferred_element_type=jnp.float32)
    out_ref[...] = hw * dinv_ref[...]


_tc_first = pl.pallas_call(
    _first_body,
    grid=(GRID,),
    in_specs=[
        pl.BlockSpec((BLK, D), lambda i: (i, 0)),
        pl.BlockSpec((D, D), lambda i: (0, 0)),
        pl.BlockSpec((BLK, 1), lambda i: (i, 0)),
    ],
    out_specs=pl.BlockSpec((BLK, D), lambda i: (i, 0)),
    out_shape=jax.ShapeDtypeStruct((N2, D), jnp.float32),
)


def _mid_body(a0_ref, a1_ref, q_ref, dinv_ref, b_ref, w_ref, out_ref):
    t = (a0_ref[...] + a1_ref[...] + q_ref[...]) * dinv_ref[...] + b_ref[...]
    h = jax.nn.sigmoid(t)
    hw = jnp.dot(h, w_ref[...], preferred_element_type=jnp.float32)
    out_ref[...] = hw * dinv_ref[...]


_tc_mid = pl.pallas_call(
    _mid_body,
    grid=(GRID,),
    in_specs=[
        pl.BlockSpec((BLK, D), lambda i: (i, 0)),
        pl.BlockSpec((BLK, D), lambda i: (i, 0)),
        pl.BlockSpec((BLK, D), lambda i: (i, 0)),
        pl.BlockSpec((BLK, 1), lambda i: (i, 0)),
        pl.BlockSpec((1, D), lambda i: (0, 0)),
        pl.BlockSpec((D, D), lambda i: (0, 0)),
    ],
    out_specs=pl.BlockSpec((BLK, D), lambda i: (i, 0)),
    out_shape=jax.ShapeDtypeStruct((N2, D), jnp.float32),
)


def _out_body(a0_ref, a1_ref, q_ref, dinv_ref, b_ref, out_ref):
    t = (a0_ref[...] + a1_ref[...] + q_ref[...]) * dinv_ref[...] + b_ref[...]
    out_ref[...] = jax.nn.sigmoid(t)


_tc_out = pl.pallas_call(
    _out_body,
    grid=(GRID,),
    in_specs=[
        pl.BlockSpec((BLK, D), lambda i: (i, 0)),
        pl.BlockSpec((BLK, D), lambda i: (i, 0)),
        pl.BlockSpec((BLK, D), lambda i: (i, 0)),
        pl.BlockSpec((BLK, 1), lambda i: (i, 0)),
        pl.BlockSpec((1, D), lambda i: (0, 0)),
    ],
    out_specs=pl.BlockSpec((BLK, D), lambda i: (i, 0)),
    out_shape=jax.ShapeDtypeStruct((N2, D), jnp.float32),
)


def kernel(x, edge_index, W1, b1, W2, b2, W3, b3, W4, b4):
    ei = edge_index.astype(jnp.int32)
    src = jnp.concatenate([ei[0], jnp.zeros((EP - E,), jnp.int32)])
    dst = jnp.concatenate([ei[1], jnp.full((EP - E,), N, jnp.int32)])
    src_b = src.reshape(TOTC, CHUNK)
    dst_b = dst.reshape(TOTC, CHUNK)

    xp = jnp.pad(x, ((0, N2 - N), (0, 0)))
    zeros1 = jnp.zeros((N2,), jnp.float32)
    zeros2 = jnp.zeros((N2, D), jnp.float32)

    deg_p = _sc_deg(dst_b, zeros1)
    dinv = _tc_dinv(deg_p.reshape(2, N2 // D, D)).reshape(N2, 1)

    q = _tc_first(xp, W1, dinv)
    for (W_next, b) in ((W2, b1), (W3, b2), (W4, b3)):
        acc = _sc_scatter(q, src_b, dst_b, zeros2)
        q = _tc_mid(acc[0], acc[1], q, dinv, b.reshape(1, D), W_next)
    acc = _sc_scatter(q, src_b, dst_b, zeros2)
    out = _tc_out(acc[0], acc[1], q, dinv, b4.reshape(1, D))
    return out[:N]


# speedup vs baseline: 1.1056x; 1.1056x over previous
import functools

import jax
import jax.numpy as jnp
from jax import lax
from jax.experimental import pallas as pl
from jax.experimental.pallas import tpu as pltpu
from jax.experimental.pallas import tpu_sc as plsc

N = 10000
E = 320000
D = 128
N2 = 10240
NW = 32
CHUNK = 128
NCH = 80
EP = NW * CHUNK * NCH
TOTC = EP // CHUNK
K0 = 80
K1 = 80
KPAIR = K0 + K1
KMAX = max(K0, K1)
SLAB = N2 // 16
BLK = 1024
GRID = N2 // BLK

_sc_mesh = plsc.VectorSubcoreMesh(core_axis_name="c", subcore_axis_name="s")


@functools.partial(
    pl.kernel, mesh=_sc_mesh,
    out_type=jax.ShapeDtypeStruct((2, N2), jnp.float32),
    scratch_types=[
        pltpu.VMEM((NCH, CHUNK), jnp.int32),
        pltpu.VMEM((CHUNK,), jnp.float32),
        pltpu.VMEM_SHARED((N2,), jnp.float32),
    ],
)
def _sc_deg(dst_hbm, zeros1_hbm, out_hbm, dst_v, ones_v, acc_sp):
    c = lax.axis_index("c")
    s = lax.axis_index("s")
    base = (s * 2 + c) * NCH
    pltpu.sync_copy(zeros1_hbm.at[pl.ds(s * SLAB, SLAB)],
                    acc_sp.at[pl.ds(s * SLAB, SLAB)])
    for l in range(CHUNK // 16):
        ones_v[pl.ds(l * 16, 16)] = jnp.ones((16,), jnp.float32)
    pltpu.sync_copy(dst_hbm.at[pl.ds(base, NCH)], dst_v)
    plsc.subcore_barrier()

    def body(j, carry):
        pltpu.sync_copy(ones_v, acc_sp.at[dst_v.at[j]], add=True)
        return carry

    lax.fori_loop(0, NCH, body, 0)
    plsc.subcore_barrier()
    pltpu.sync_copy(acc_sp.at[pl.ds(s * SLAB, SLAB)],
                    out_hbm.at[c, pl.ds(s * SLAB, SLAB)])


_sc1_mesh = plsc.VectorSubcoreMesh(core_axis_name="c", subcore_axis_name="s",
                                   num_cores=1)
HALFC = TOTC // 2
KSUB = HALFC // 16


@functools.partial(
    pl.kernel, mesh=_sc1_mesh,
    out_type=jax.ShapeDtypeStruct((N2, D), jnp.float32),
    scratch_types=[
        pltpu.VMEM((KSUB, CHUNK), jnp.int32),
        pltpu.VMEM((KSUB, CHUNK), jnp.int32),
        pltpu.VMEM((CHUNK, D), jnp.float32),
        pltpu.VMEM_SHARED((N2, D), jnp.float32),
        pltpu.SemaphoreType.DMA,
    ],
)
def _sc_scatter_half(q_hbm, src_hbm, dst_hbm, zeros2_hbm, out_hbm,
                     src_v, dst_v, rows_v, acc_sp, sem):
    s = lax.axis_index("s")
    base = s * KSUB
    pltpu.sync_copy(zeros2_hbm.at[pl.ds(s * SLAB, SLAB)],
                    acc_sp.at[pl.ds(s * SLAB, SLAB)])
    pltpu.sync_copy(src_hbm.at[pl.ds(base, KSUB)], src_v)
    pltpu.sync_copy(dst_hbm.at[pl.ds(base, KSUB)], dst_v)
    plsc.subcore_barrier()

    def body(j, carry):
        pltpu.async_copy(q_hbm.at[src_v.at[j]], rows_v, sem).wait()
        pltpu.sync_copy(rows_v, acc_sp.at[dst_v.at[j]], add=True)
        return carry

    lax.fori_loop(0, KSUB, body, 0)
    plsc.subcore_barrier()
    pltpu.sync_copy(acc_sp.at[pl.ds(s * SLAB, SLAB)],
                    out_hbm.at[pl.ds(s * SLAB, SLAB)])


def _dinv_body(deg_ref, out_ref):
    d = deg_ref[0] + deg_ref[1] + 1.0
    out_ref[...] = lax.rsqrt(d)


_tc_dinv = pl.pallas_call(
    _dinv_body,
    out_shape=jax.ShapeDtypeStruct((N2 // D, D), jnp.float32),
)


def _first_body(x_ref, w_ref, dinv_ref, out_ref):
    hw = jnp.dot(x_ref[...], w_ref[...], preferred_element_type=jnp.float32)
    out_ref[...] = hw * dinv_ref[...]


_tc_first = pl.pallas_call(
    _first_body,
    grid=(GRID,),
    in_specs=[
        pl.BlockSpec((BLK, D), lambda i: (i, 0)),
        pl.BlockSpec((D, D), lambda i: (0, 0)),
        pl.BlockSpec((BLK, 1), lambda i: (i, 0)),
    ],
    out_specs=pl.BlockSpec((BLK, D), lambda i: (i, 0)),
    out_shape=jax.ShapeDtypeStruct((N2, D), jnp.float32),
)


def _mid_body(a0_ref, a1_ref, q_ref, dinv_ref, b_ref, w_ref, out_ref):
    t = (a0_ref[...] + a1_ref[...] + q_ref[...]) * dinv_ref[...] + b_ref[...]
    h = jax.nn.sigmoid(t)
    hw = jnp.dot(h, w_ref[...], preferred_element_type=jnp.float32)
    out_ref[...] = hw * dinv_ref[...]


_tc_mid = pl.pallas_call(
    _mid_body,
    grid=(GRID,),
    in_specs=[
        pl.BlockSpec((BLK, D), lambda i: (i, 0)),
        pl.BlockSpec((BLK, D), lambda i: (i, 0)),
        pl.BlockSpec((BLK, D), lambda i: (i, 0)),
        pl.BlockSpec((BLK, 1), lambda i: (i, 0)),
        pl.BlockSpec((1, D), lambda i: (0, 0)),
        pl.BlockSpec((D, D), lambda i: (0, 0)),
    ],
    out_specs=pl.BlockSpec((BLK, D), lambda i: (i, 0)),
    out_shape=jax.ShapeDtypeStruct((N2, D), jnp.float32),
)


def _out_body(a0_ref, a1_ref, q_ref, dinv_ref, b_ref, out_ref):
    t = (a0_ref[...] + a1_ref[...] + q_ref[...]) * dinv_ref[...] + b_ref[...]
    out_ref[...] = jax.nn.sigmoid(t)


_tc_out = pl.pallas_call(
    _out_body,
    grid=(GRID,),
    in_specs=[
        pl.BlockSpec((BLK, D), lambda i: (i, 0)),
        pl.BlockSpec((BLK, D), lambda i: (i, 0)),
        pl.BlockSpec((BLK, D), lambda i: (i, 0)),
        pl.BlockSpec((BLK, 1), lambda i: (i, 0)),
        pl.BlockSpec((1, D), lambda i: (0, 0)),
    ],
    out_specs=pl.BlockSpec((BLK, D), lambda i: (i, 0)),
    out_shape=jax.ShapeDtypeStruct((N2, D), jnp.float32),
)


def kernel(x, edge_index, W1, b1, W2, b2, W3, b3, W4, b4):
    ei = edge_index.astype(jnp.int32)
    src = jnp.concatenate([ei[0], jnp.zeros((EP - E,), jnp.int32)])
    dst = jnp.concatenate([ei[1], jnp.full((EP - E,), N, jnp.int32)])
    src_b = src.reshape(TOTC, CHUNK)
    dst_b = dst.reshape(TOTC, CHUNK)

    xp = jnp.pad(x, ((0, N2 - N), (0, 0)))
    zeros1 = jnp.zeros((N2,), jnp.float32)
    zeros2 = jnp.zeros((N2, D), jnp.float32)

    deg_p = _sc_deg(dst_b, zeros1)
    dinv = _tc_dinv(deg_p.reshape(2, N2 // D, D)).reshape(N2, 1)

    src_a, src_c = src_b[:HALFC], src_b[HALFC:]
    dst_a, dst_c = dst_b[:HALFC], dst_b[HALFC:]

    def edge_scatter(q):
        acc_a = _sc_scatter_half(q, src_a, dst_a, zeros2)
        acc_b = _sc_scatter_half(q, src_c, dst_c, zeros2)
        return acc_a, acc_b

    q = _tc_first(xp, W1, dinv)
    for (W_next, b) in ((W2, b1), (W3, b2), (W4, b3)):
        acc_a, acc_b = edge_scatter(q)
        q = _tc_mid(acc_a, acc_b, q, dinv, b.reshape(1, D), W_next)
    acc_a, acc_b = edge_scatter(q)
    out = _tc_out(acc_a, acc_b, q, dinv, b4.reshape(1, D))
    return out[:N]


# speedup vs baseline: 1.5181x; 1.3731x over previous
import functools

import jax
import jax.numpy as jnp
from jax import lax
from jax.experimental import pallas as pl
from jax.experimental.pallas import tpu as pltpu
from jax.experimental.pallas import tpu_sc as plsc

N = 10000
E = 320000
D = 128
N2 = 10240
NW = 32
CHUNK = 128
NCH = 80
EP = NW * CHUNK * NCH
TOTC = EP // CHUNK
K0 = 80
K1 = 80
KPAIR = K0 + K1
KMAX = max(K0, K1)
SLAB = N2 // 16
BLK = 1024
GRID = N2 // BLK

_sc_mesh = plsc.VectorSubcoreMesh(core_axis_name="c", subcore_axis_name="s")


@functools.partial(
    pl.kernel, mesh=_sc_mesh,
    out_type=jax.ShapeDtypeStruct((2, N2), jnp.float32),
    scratch_types=[
        pltpu.VMEM((NCH, CHUNK), jnp.int32),
        pltpu.VMEM((CHUNK,), jnp.float32),
        pltpu.VMEM_SHARED((N2,), jnp.float32),
    ],
)
def _sc_deg(dst_hbm, zeros1_hbm, out_hbm, dst_v, ones_v, acc_sp):
    c = lax.axis_index("c")
    s = lax.axis_index("s")
    base = (s * 2 + c) * NCH
    pltpu.sync_copy(zeros1_hbm.at[pl.ds(s * SLAB, SLAB)],
                    acc_sp.at[pl.ds(s * SLAB, SLAB)])
    for l in range(CHUNK // 16):
        ones_v[pl.ds(l * 16, 16)] = jnp.ones((16,), jnp.float32)
    pltpu.sync_copy(dst_hbm.at[pl.ds(base, NCH)], dst_v)
    plsc.subcore_barrier()

    def body(j, carry):
        pltpu.sync_copy(ones_v, acc_sp.at[dst_v.at[j]], add=True)
        return carry

    lax.fori_loop(0, NCH, body, 0)
    plsc.subcore_barrier()
    pltpu.sync_copy(acc_sp.at[pl.ds(s * SLAB, SLAB)],
                    out_hbm.at[c, pl.ds(s * SLAB, SLAB)])


@functools.partial(
    pl.kernel, mesh=_sc_mesh,
    out_type=jax.ShapeDtypeStruct((2, N2, D), jnp.float32),
    scratch_types=[
        pltpu.VMEM((NCH, CHUNK), jnp.int32),
        pltpu.VMEM((NCH, CHUNK), jnp.int32),
        pltpu.VMEM((CHUNK, D), jnp.float32),
        pltpu.VMEM_SHARED((N2, D), jnp.float32),
        pltpu.SemaphoreType.DMA,
    ],
)
def _sc_scatter(q_hbm, src_hbm, dst_hbm, zeros2_hbm, out_hbm,
                src_v, dst_v, rows_v, acc_sp, sem):
    c = lax.axis_index("c")
    s = lax.axis_index("s")
    base = (s * 2 + c) * NCH
    pltpu.sync_copy(zeros2_hbm.at[pl.ds(s * SLAB, SLAB)],
                    acc_sp.at[pl.ds(s * SLAB, SLAB)])
    pltpu.sync_copy(src_hbm.at[pl.ds(base, NCH)], src_v)
    pltpu.sync_copy(dst_hbm.at[pl.ds(base, NCH)], dst_v)
    plsc.subcore_barrier()

    def body(i, carry):
        j = i * 2
        pltpu.async_copy(q_hbm.at[src_v.at[j]], rows_v, sem).wait()
        pltpu.sync_copy(rows_v, acc_sp.at[dst_v.at[j]], add=True)
        pltpu.async_copy(q_hbm.at[src_v.at[j + 1]], rows_v, sem).wait()
        pltpu.sync_copy(rows_v, acc_sp.at[dst_v.at[j + 1]], add=True)
        return carry

    lax.fori_loop(0, NCH // 2, body, 0)
    plsc.subcore_barrier()
    pltpu.sync_copy(acc_sp.at[pl.ds(s * SLAB, SLAB)],
                    out_hbm.at[c, pl.ds(s * SLAB, SLAB)])


def _dinv_body(deg_ref, out_ref):
    d = deg_ref[0] + deg_ref[1] + 1.0
    out_ref[...] = lax.rsqrt(d)


_tc_dinv = pl.pallas_call(
    _dinv_body,
    out_shape=jax.ShapeDtypeStruct((N2 // D, D), jnp.float32),
)


def _first_body(x_ref, w_ref, dinv_ref, out_ref):
    hw = jnp.dot(x_ref[...], w_ref[...], preferred_element_type=jnp.float32)
    out_ref[...] = hw * dinv_ref[...]


_tc_first = pl.pallas_call(
    _first_body,
    grid=(GRID,),
    in_specs=[
        pl.BlockSpec((BLK, D), lambda i: (i, 0)),
        pl.BlockSpec((D, D), lambda i: (0, 0)),
        pl.BlockSpec((BLK, 1), lambda i: (i, 0)),
    ],
    out_specs=pl.BlockSpec((BLK, D), lambda i: (i, 0)),
    out_shape=jax.ShapeDtypeStruct((N2, D), jnp.float32),
)


def _mid_body(a0_ref, a1_ref, q_ref, dinv_ref, b_ref, w_ref, out_ref):
    t = (a0_ref[...] + a1_ref[...] + q_ref[...]) * dinv_ref[...] + b_ref[...]
    h = jax.nn.sigmoid(t)
    hw = jnp.dot(h, w_ref[...], preferred_element_type=jnp.float32)
    out_ref[...] = hw * dinv_ref[...]


_tc_mid = pl.pallas_call(
    _mid_body,
    grid=(GRID,),
    in_specs=[
        pl.BlockSpec((BLK, D), lambda i: (i, 0)),
        pl.BlockSpec((BLK, D), lambda i: (i, 0)),
        pl.BlockSpec((BLK, D), lambda i: (i, 0)),
        pl.BlockSpec((BLK, 1), lambda i: (i, 0)),
        pl.BlockSpec((1, D), lambda i: (0, 0)),
        pl.BlockSpec((D, D), lambda i: (0, 0)),
    ],
    out_specs=pl.BlockSpec((BLK, D), lambda i: (i, 0)),
    out_shape=jax.ShapeDtypeStruct((N2, D), jnp.float32),
)


def _out_body(a0_ref, a1_ref, q_ref, dinv_ref, b_ref, out_ref):
    t = (a0_ref[...] + a1_ref[...] + q_ref[...]) * dinv_ref[...] + b_ref[...]
    out_ref[...] = jax.nn.sigmoid(t)


_tc_out = pl.pallas_call(
    _out_body,
    grid=(GRID,),
    in_specs=[
        pl.BlockSpec((BLK, D), lambda i: (i, 0)),
        pl.BlockSpec((BLK, D), lambda i: (i, 0)),
        pl.BlockSpec((BLK, D), lambda i: (i, 0)),
        pl.BlockSpec((BLK, 1), lambda i: (i, 0)),
        pl.BlockSpec((1, D), lambda i: (0, 0)),
    ],
    out_specs=pl.BlockSpec((BLK, D), lambda i: (i, 0)),
    out_shape=jax.ShapeDtypeStruct((N2, D), jnp.float32),
)


def kernel(x, edge_index, W1, b1, W2, b2, W3, b3, W4, b4):
    ei = edge_index.astype(jnp.int32)
    src = jnp.concatenate([ei[0], jnp.zeros((EP - E,), jnp.int32)])
    dst = jnp.concatenate([ei[1], jnp.full((EP - E,), N, jnp.int32)])
    src_b = src.reshape(TOTC, CHUNK)
    dst_b = dst.reshape(TOTC, CHUNK)

    xp = jnp.pad(x, ((0, N2 - N), (0, 0)))
    zeros1 = jnp.zeros((N2,), jnp.float32)
    zeros2 = jnp.zeros((N2, D), jnp.float32)

    deg_p = _sc_deg(dst_b, zeros1)
    dinv = _tc_dinv(deg_p.reshape(2, N2 // D, D)).reshape(N2, 1)

    q = _tc_first(xp, W1, dinv)
    for (W_next, b) in ((W2, b1), (W3, b2), (W4, b3)):
        acc = _sc_scatter(q, src_b, dst_b, zeros2)
        q = _tc_mid(acc[0], acc[1], q, dinv, b.reshape(1, D), W_next)
    acc = _sc_scatter(q, src_b, dst_b, zeros2)
    out = _tc_out(acc[0], acc[1], q, dinv, b4.reshape(1, D))
    return out[:N]


# speedup vs baseline: 1.9830x; 1.3062x over previous
import functools

import jax
import jax.numpy as jnp
from jax import lax
from jax.experimental import pallas as pl
from jax.experimental.pallas import tpu as pltpu
from jax.experimental.pallas import tpu_sc as plsc

N = 10000
E = 320000
D = 128
N2 = 10240
NW = 32
CHUNK = 128
NCH = -(-E // (NW * CHUNK))
EP = NW * CHUNK * NCH
SLAB = N2 // 16
BLK = 1024
GRID = N2 // BLK

_sc_mesh = plsc.VectorSubcoreMesh(core_axis_name="c", subcore_axis_name="s")


@functools.partial(
    pl.kernel, mesh=_sc_mesh,
    out_type=jax.ShapeDtypeStruct((2, N2), jnp.float32),
    scratch_types=[
        pltpu.VMEM((NCH, CHUNK), jnp.int32),
        pltpu.VMEM((CHUNK,), jnp.float32),
        pltpu.VMEM_SHARED((N2,), jnp.float32),
    ],
)
def _sc_deg(dst_hbm, zeros1_hbm, out_hbm, dst_v, ones_v, acc_sp):
    c = lax.axis_index("c")
    s = lax.axis_index("s")
    wid = s * 2 + c
    pltpu.sync_copy(zeros1_hbm.at[pl.ds(s * SLAB, SLAB)],
                    acc_sp.at[pl.ds(s * SLAB, SLAB)])
    for l in range(CHUNK // 16):
        ones_v[pl.ds(l * 16, 16)] = jnp.ones((16,), jnp.float32)
    pltpu.sync_copy(dst_hbm.at[wid], dst_v)
    plsc.subcore_barrier()

    def body(j, carry):
        pltpu.sync_copy(ones_v, acc_sp.at[dst_v.at[j]], add=True)
        return carry

    lax.fori_loop(0, NCH, body, 0)
    plsc.subcore_barrier()
    pltpu.sync_copy(acc_sp.at[pl.ds(s * SLAB, SLAB)],
                    out_hbm.at[c, pl.ds(s * SLAB, SLAB)])


@functools.partial(
    pl.kernel, mesh=_sc_mesh,
    out_type=jax.ShapeDtypeStruct((2, N2, D), jnp.float32),
    scratch_types=[
        pltpu.VMEM((NCH, CHUNK), jnp.int32),
        pltpu.VMEM((NCH, CHUNK), jnp.int32),
        pltpu.VMEM((CHUNK, D), jnp.float32),
        pltpu.VMEM_SHARED((N2, D), jnp.float32),
        pltpu.SemaphoreType.DMA,
    ],
)
def _sc_scatter(q_hbm, src_hbm, dst_hbm, zeros2_hbm, out_hbm,
                src_v, dst_v, rows_v, acc_sp, sem):
    c = lax.axis_index("c")
    s = lax.axis_index("s")
    wid = s * 2 + c
    pltpu.sync_copy(zeros2_hbm.at[pl.ds(s * SLAB, SLAB)],
                    acc_sp.at[pl.ds(s * SLAB, SLAB)])
    pltpu.sync_copy(src_hbm.at[wid], src_v)
    pltpu.sync_copy(dst_hbm.at[wid], dst_v)
    plsc.subcore_barrier()

    def body(j, carry):
        pltpu.async_copy(q_hbm.at[src_v.at[j]], rows_v, sem).wait()
        pltpu.sync_copy(rows_v, acc_sp.at[dst_v.at[j]], add=True)
        return carry

    lax.fori_loop(0, NCH, body, 0)
    plsc.subcore_barrier()
    pltpu.sync_copy(acc_sp.at[pl.ds(s * SLAB, SLAB)],
                    out_hbm.at[c, pl.ds(s * SLAB, SLAB)])


def _dinv_body(deg_ref, out_ref):
    d = deg_ref[0] + deg_ref[1] + 1.0
    out_ref[...] = lax.rsqrt(d)


_tc_dinv = pl.pallas_call(
    _dinv_body,
    out_shape=jax.ShapeDtypeStruct((N2 // D, D), jnp.float32),
)


def _first_body(x_ref, w_ref, dinv_ref, out_ref):
    hw = jnp.dot(x_ref[...], w_ref[...], preferred_element_type=jnp.float32)
    out_ref[...] = hw * dinv_ref[...]


_tc_first = pl.pallas_call(
    _first_body,
    grid=(GRID,),
    in_specs=[
        pl.BlockSpec((BLK, D), lambda i: (i, 0)),
        pl.BlockSpec((D, D), lambda i: (0, 0)),
        pl.BlockSpec((BLK, 1), lambda i: (i, 0)),
    ],
    out_specs=pl.BlockSpec((BLK, D), lambda i: (i, 0)),
    out_shape=jax.ShapeDtypeStruct((N2, D), jnp.float32),
)


def _mid_body(a0_ref, a1_ref, q_ref, dinv_ref, b_ref, w_ref, out_ref):
    t = (a0_ref[...] + a1_ref[...] + q_ref[...]) * dinv_ref[...] + b_ref[...]
    h = jax.nn.sigmoid(t)
    hw = jnp.dot(h, w_ref[...], preferred_element_type=jnp.float32)
    out_ref[...] = hw * dinv_ref[...]


_tc_mid = pl.pallas_call(
    _mid_body,
    grid=(GRID,),
    in_specs=[
        pl.BlockSpec((BLK, D), lambda i: (i, 0)),
        pl.BlockSpec((BLK, D), lambda i: (i, 0)),
        pl.BlockSpec((BLK, D), lambda i: (i, 0)),
        pl.BlockSpec((BLK, 1), lambda i: (i, 0)),
        pl.BlockSpec((1, D), lambda i: (0, 0)),
        pl.BlockSpec((D, D), lambda i: (0, 0)),
    ],
    out_specs=pl.BlockSpec((BLK, D), lambda i: (i, 0)),
    out_shape=jax.ShapeDtypeStruct((N2, D), jnp.float32),
)


def _out_body(a0_ref, a1_ref, q_ref, dinv_ref, b_ref, out_ref):
    t = (a0_ref[...] + a1_ref[...] + q_ref[...]) * dinv_ref[...] + b_ref[...]
    out_ref[...] = jax.nn.sigmoid(t)


_tc_out = pl.pallas_call(
    _out_body,
    grid=(GRID,),
    in_specs=[
        pl.BlockSpec((BLK, D), lambda i: (i, 0)),
        pl.BlockSpec((BLK, D), lambda i: (i, 0)),
        pl.BlockSpec((BLK, D), lambda i: (i, 0)),
        pl.BlockSpec((BLK, 1), lambda i: (i, 0)),
        pl.BlockSpec((1, D), lambda i: (0, 0)),
    ],
    out_specs=pl.BlockSpec((BLK, D), lambda i: (i, 0)),
    out_shape=jax.ShapeDtypeStruct((N2, D), jnp.float32),
)


def kernel(x, edge_index, W1, b1, W2, b2, W3, b3, W4, b4):
    ei = edge_index.astype(jnp.int32)
    src = jnp.concatenate([ei[0], jnp.zeros((EP - E,), jnp.int32)])
    dst = jnp.concatenate([ei[1], jnp.full((EP - E,), N, jnp.int32)])
    src_b = src.reshape(NW, NCH, CHUNK)
    dst_b = dst.reshape(NW, NCH, CHUNK)

    xp = jnp.pad(x, ((0, N2 - N), (0, 0)))
    zeros1 = jnp.zeros((N2,), jnp.float32)
    zeros2 = jnp.zeros((N2, D), jnp.float32)

    deg_p = _sc_deg(dst_b, zeros1)
    dinv = _tc_dinv(deg_p.reshape(2, N2 // D, D)).reshape(N2, 1)

    q = _tc_first(xp, W1, dinv)
    for (W_next, b) in ((W2, b1), (W3, b2), (W4, b3)):
        acc = _sc_scatter(q, src_b, dst_b, zeros2)
        q = _tc_mid(acc[0], acc[1], q, dinv, b.reshape(1, D), W_next)
    acc = _sc_scatter(q, src_b, dst_b, zeros2)
    out = _tc_out(acc[0], acc[1], q, dinv, b4.reshape(1, D))
    return out[:N]
